# baseline (speedup 1.0000x reference)
import jax
import jax.numpy as jnp
from jax.experimental import pallas as pl
from jax.experimental.pallas import tpu as pltpu

_ALPHA = 0.1
_BLOCK = 2000
_SUPER = 4


def _appnp_block(x_ref, agg_ref, h_ref, nb_ref, x_out_ref, agg_out_ref):
    a = _ALPHA
    i = pl.program_id(0)
    r0 = (i % _SUPER) * _BLOCK
    agg = agg_ref[...]
    s = jnp.sum(agg, axis=0)
    xs = x_ref[pl.ds(r0, _BLOCK), :]
    hs = h_ref[pl.ds(r0, _BLOCK), :]
    x_out_ref[...] = (1.0 - a) * (xs + s) + a * hs
    agg_out_ref[...] = (1.0 - a) * agg + a * nb_ref[...]


@jax.jit
def kernel(x, neighbor_agg, h, neighbor):
    n, d = x.shape
    k = neighbor_agg.shape[0]
    blk = _BLOCK
    grid = (n // blk,)

    super_spec = pl.BlockSpec((_SUPER * blk, d), lambda i: (i // _SUPER, 0))
    row_spec = pl.BlockSpec((blk, d), lambda i: (i, 0))
    hop_spec = pl.BlockSpec((k, blk, d), lambda i: (0, i, 0))

    return pl.pallas_call(
        _appnp_block,
        grid=grid,
        in_specs=[super_spec, hop_spec, super_spec, hop_spec],
        out_specs=[row_spec, hop_spec],
        out_shape=[
            jax.ShapeDtypeStruct((n, d), x.dtype),
            jax.ShapeDtypeStruct((k, n, d), neighbor_agg.dtype),
        ],
        compiler_params=pltpu.CompilerParams(
            dimension_semantics=("arbitrary",),
        ),
    )(x, neighbor_agg, h, neighbor)
